# baseline (speedup 1.0000x reference)
import functools

import jax
import jax.numpy as jnp
from jax.experimental import pallas as pl
from jax.experimental.pallas import tpu as pltpu

_GBLK = 128
_CBLK = 8192


def _gather_body(t_ref, *refs, bsz):
    xs, out_ref = refs[:bsz], refs[bsz]
    lane = jax.lax.broadcasted_iota(jnp.int32, (1, 1, _GBLK), 2)
    for r in range(bsz):
        t = t_ref[r]
        loc = t - (t // _GBLK) * _GBLK
        out_ref[r] = jnp.sum(jnp.where(lane == loc, xs[r][...], 0.0))


def _count_body(x_ref, v_ref, t_ref, out1_ref, out5_ref, acc_ref, *, nsteps, n, bsz):
    j = pl.program_id(0)

    @pl.when(j == 0)
    def _init():
        acc_ref[...] = jnp.zeros_like(acc_ref)

    x = x_ref[...]
    v = v_ref[...]
    tloc = t_ref[...] - j * _CBLK
    lane = jax.lax.broadcasted_iota(jnp.int32, x.shape, 1)
    beat = (x > v) | ((x == v) & (lane < tloc))

    @pl.when(j < n // _CBLK)
    def _full():
        acc_ref[...] += beat.astype(jnp.int32)

    @pl.when(j == nsteps - 1)
    def _fin():
        if n % _CBLK:
            acc_ref[...] += (beat & (lane < (n - (nsteps - 1) * _CBLK))).astype(jnp.int32)
        rank = jnp.sum(acc_ref[...], axis=1, keepdims=True)
        scale = 100.0 / bsz
        out1_ref[...] = jnp.sum((rank < 1).astype(jnp.float32),
                                axis=(0, 1), keepdims=True) * scale
        out5_ref[...] = jnp.sum((rank < 5).astype(jnp.float32),
                                axis=(0, 1), keepdims=True) * scale


def kernel(preds, targets):
    bsz, n = preds.shape
    t32 = targets.astype(jnp.int32)
    preds3 = preds.reshape(bsz, 1, n)

    def _mk_spec(r):
        return pl.BlockSpec((1, 1, _GBLK), lambda i, t, r=r: (r, 0, t[r] // _GBLK))

    v = pl.pallas_call(
        functools.partial(_gather_body, bsz=bsz),
        grid_spec=pltpu.PrefetchScalarGridSpec(
            num_scalar_prefetch=1,
            grid=(1,),
            in_specs=[_mk_spec(r) for r in range(bsz)],
            out_specs=pl.BlockSpec(memory_space=pltpu.SMEM),
        ),
        out_shape=jax.ShapeDtypeStruct((bsz,), jnp.float32),
    )(t32, *([preds3] * bsz))

    nsteps = pl.cdiv(n, _CBLK)
    out1, out5 = pl.pallas_call(
        functools.partial(_count_body, nsteps=nsteps, n=n, bsz=bsz),
        grid=(nsteps,),
        in_specs=[
            pl.BlockSpec((bsz, _CBLK), lambda j: (0, j)),
            pl.BlockSpec((bsz, 1), lambda j: (0, 0)),
            pl.BlockSpec((bsz, 1), lambda j: (0, 0)),
        ],
        out_specs=[
            pl.BlockSpec((1, 1), lambda j: (0, 0)),
            pl.BlockSpec((1, 1), lambda j: (0, 0)),
        ],
        out_shape=[jax.ShapeDtypeStruct((1, 1), jnp.float32)] * 2,
        scratch_shapes=[pltpu.VMEM((bsz, _CBLK), jnp.int32)],
    )(preds, v.reshape(bsz, 1), t32.reshape(bsz, 1))

    return (out1.reshape(1), out5.reshape(1))


# speedup vs baseline: 2.4525x; 2.4525x over previous
import functools

import jax
import jax.numpy as jnp
from jax.experimental import pallas as pl
from jax.experimental.pallas import tpu as pltpu

_RB = 8
_GW = 128


def _body(tstart_ref, p_ref, x_ref, tmod_ref, t_ref,
          out1_ref, out5_ref, v_scr, x_scr, rank_scr, sem, *, nsteps, bsz):
    j = pl.program_id(0)

    @pl.when(j == 0)
    def _gather():
        copies = [
            pltpu.make_async_copy(
                p_ref.at[pl.ds(8 * (r // 8), 8),
                         pl.ds(pl.multiple_of(tstart_ref[r], _GW), _GW)],
                x_scr.at[r],
                sem,
            )
            for r in range(bsz)
        ]
        for c in copies:
            c.start()
        for c in copies:
            c.wait()
        sub = jax.lax.broadcasted_iota(jnp.int32, (bsz, 8, _GW), 1)
        rmod = jax.lax.broadcasted_iota(jnp.int32, (bsz, 8, _GW), 0) % 8
        lane = jax.lax.broadcasted_iota(jnp.int32, (bsz, 8, _GW), 2)
        sel = jnp.where((sub == rmod) & (lane == tmod_ref[...]),
                        x_scr[...], 0.0)
        v_scr[...] = jnp.sum(sel, axis=(1, 2)).reshape(bsz, 1)

    x = x_ref[...]
    v = v_scr[pl.ds(j * _RB, _RB), :]
    t = t_ref[...]
    lane = jax.lax.broadcasted_iota(jnp.int32, x.shape, 1)
    beat = (x > v) | ((x == v) & (lane < t))
    rank_scr[pl.ds(j * _RB, _RB), :] = jnp.sum(beat.astype(jnp.int32),
                                               axis=1, keepdims=True)

    @pl.when(j == nsteps - 1)
    def _fin():
        rank = rank_scr[...]
        scale = 100.0 / bsz
        out1_ref[...] = jnp.sum((rank < 1).astype(jnp.float32),
                                axis=(0, 1), keepdims=True) * scale
        out5_ref[...] = jnp.sum((rank < 5).astype(jnp.float32),
                                axis=(0, 1), keepdims=True) * scale


def kernel(preds, targets):
    bsz, n = preds.shape
    t32 = targets.astype(jnp.int32)
    tstart = (t32 // _GW) * _GW
    tmod = (t32 % _GW).reshape(bsz, 1, 1)

    nsteps = bsz // _RB
    out1, out5 = pl.pallas_call(
        functools.partial(_body, nsteps=nsteps, bsz=bsz),
        grid_spec=pltpu.PrefetchScalarGridSpec(
            num_scalar_prefetch=1,
            grid=(nsteps,),
            in_specs=[
                pl.BlockSpec(memory_space=pl.ANY),
                pl.BlockSpec((_RB, n), lambda j, s: (j, 0)),
                pl.BlockSpec((bsz, 1, 1), lambda j, s: (0, 0, 0)),
                pl.BlockSpec((_RB, 1), lambda j, s: (j, 0)),
            ],
            out_specs=[
                pl.BlockSpec((1, 1), lambda j, s: (0, 0)),
                pl.BlockSpec((1, 1), lambda j, s: (0, 0)),
            ],
            scratch_shapes=[
                pltpu.VMEM((bsz, 1), jnp.float32),
                pltpu.VMEM((bsz, 8, _GW), jnp.float32),
                pltpu.VMEM((bsz, 1), jnp.int32),
                pltpu.SemaphoreType.DMA,
            ],
        ),
        out_shape=[jax.ShapeDtypeStruct((1, 1), jnp.float32)] * 2,
    )(tstart, preds, preds, tmod, t32.reshape(bsz, 1))

    return (out1.reshape(1), out5.reshape(1))


# speedup vs baseline: 2.5480x; 1.0389x over previous
import functools

import jax
import jax.numpy as jnp
from jax.experimental import pallas as pl
from jax.experimental.pallas import tpu as pltpu

_RB = 8
_GW = 128
_NREF = 4


def _body(tstart_ref, p_ref, *refs, nsteps, bsz):
    (x_refs, (tmod_ref, t_ref), (out1_ref, out5_ref),
     (v_scr, x_scr, rank_scr, sem)) = (refs[:_NREF], refs[_NREF:_NREF + 2],
                                       refs[_NREF + 2:_NREF + 4],
                                       refs[_NREF + 4:])
    j = pl.program_id(0)

    @pl.when(j == 0)
    def _gather():
        copies = [
            pltpu.make_async_copy(
                p_ref.at[pl.ds(8 * (r // 8), 8),
                         pl.ds(pl.multiple_of(tstart_ref[r], _GW), _GW)],
                x_scr.at[r],
                sem,
            )
            for r in range(bsz)
        ]
        for c in copies:
            c.start()
        for c in copies:
            c.wait()
        sub = jax.lax.broadcasted_iota(jnp.int32, (bsz, 8, _GW), 1)
        rmod = jax.lax.broadcasted_iota(jnp.int32, (bsz, 8, _GW), 0) % 8
        lane = jax.lax.broadcasted_iota(jnp.int32, (bsz, 8, _GW), 2)
        sel = jnp.where((sub == rmod) & (lane == tmod_ref[...]),
                        x_scr[...], 0.0)
        v_scr[...] = jnp.sum(sel, axis=(1, 2)).reshape(bsz, 1)

    for r, xr in enumerate(x_refs):
        g = j * _NREF + r
        x = xr[...]
        v = v_scr[pl.ds(g * _RB, _RB), :]
        t = t_ref[pl.ds(g * _RB, _RB), :]
        lane = jax.lax.broadcasted_iota(jnp.int32, x.shape, 1)
        beat = (x > v) | ((x == v) & (lane < t))
        rank_scr[pl.ds(g * _RB, _RB), :] = jnp.sum(beat.astype(jnp.int32),
                                                   axis=1, keepdims=True)

    @pl.when(j == nsteps - 1)
    def _fin():
        rank = rank_scr[...]
        scale = 100.0 / bsz
        out1_ref[...] = jnp.sum((rank < 1).astype(jnp.float32),
                                axis=(0, 1), keepdims=True) * scale
        out5_ref[...] = jnp.sum((rank < 5).astype(jnp.float32),
                                axis=(0, 1), keepdims=True) * scale


def kernel(preds, targets):
    bsz, n = preds.shape
    t32 = targets.astype(jnp.int32)
    tstart = (t32 // _GW) * _GW
    tmod = (t32 % _GW).reshape(bsz, 1, 1)

    nsteps = bsz // (_RB * _NREF)
    out1, out5 = pl.pallas_call(
        functools.partial(_body, nsteps=nsteps, bsz=bsz),
        grid_spec=pltpu.PrefetchScalarGridSpec(
            num_scalar_prefetch=1,
            grid=(nsteps,),
            in_specs=[
                pl.BlockSpec(memory_space=pl.ANY),
            ] + [
                pl.BlockSpec((_RB, n), lambda j, s, r=r: (j * _NREF + r, 0))
                for r in range(_NREF)
            ] + [
                pl.BlockSpec((bsz, 1, 1), lambda j, s: (0, 0, 0)),
                pl.BlockSpec((bsz, 1), lambda j, s: (0, 0)),
            ],
            out_specs=[
                pl.BlockSpec((1, 1), lambda j, s: (0, 0)),
                pl.BlockSpec((1, 1), lambda j, s: (0, 0)),
            ],
            scratch_shapes=[
                pltpu.VMEM((bsz, 1), jnp.float32),
                pltpu.VMEM((bsz, 8, _GW), jnp.float32),
                pltpu.VMEM((bsz, 1), jnp.int32),
                pltpu.SemaphoreType.DMA,
            ],
        ),
        out_shape=[jax.ShapeDtypeStruct((1, 1), jnp.float32)] * 2,
    )(tstart, preds, *([preds] * _NREF), tmod, t32.reshape(bsz, 1))

    return (out1.reshape(1), out5.reshape(1))
